# baseline (speedup 1.0000x reference)
import jax
import jax.numpy as jnp
from jax import lax
from jax.experimental import pallas as pl
from jax.experimental.pallas import tpu as pltpu
from jax.experimental.pallas import tpu_sc as plsc

BATCH = 16384
N_FIELDS = 26
DIM = 32
V = 1000000
WIN = 512
NWIN = V // WIN
V_TAIL = V - NWIN * WIN
SCR_ROWS = V // 4
WLINES = WIN // 4

NC = 2
NS = 16
NWK = NC * NS

B_PER_W = BATCH // NWK
UNIT = 256
N_UNITS = N_FIELDS * 2

def _iota16():
    return lax.iota(jnp.int32, 16)


def _transpose_window(inbuf, outbuf):
    iota = _iota16()
    vc0 = iota
    vc1 = iota + 16
    zero = iota * 0

    @plsc.parallel_loop(0, WLINES, 1, unroll=8)
    def krow(k):
        vk = zero + 4 * k
        for j in range(8):
            m = j // 2
            vc = vc1 if (j % 2) else vc0
            v = plsc.load_gather(inbuf, [vc, vk + m])
            outbuf[k, pl.ds(m * 32 + (j % 2) * 16, 16)] = v


def _pack_body(emb_t, tailc, scratch, in_a, in_b, out_a, out_b, tailv,
               s_ia, s_ib, s_oa, s_ob):
    wid = lax.axis_index("s") * NC + lax.axis_index("c")
    last = NWIN - 1

    def win(t):
        return jnp.minimum(wid + NWK * t, last)

    def src(t):
        g = win(t)
        return emb_t.at[:, pl.ds(g * WIN, WIN)]

    def dst(t):
        g = win(t)
        return scratch.at[pl.ds(g * WLINES, WLINES), :]

    pltpu.sync_copy(tailc, tailv)
    pltpu.sync_copy(tailv, scratch.at[pl.ds(SCR_ROWS - 16, 16), :])

    n_t = 62
    pltpu.async_copy(src(0), in_a, s_ia)
    pltpu.async_copy(src(1), in_b, s_ib)

    def half(t, inbuf, outbuf, s_in, s_out, first, issue_next):
        if not first:
            pltpu.make_async_copy(outbuf, dst(t - 2), s_out).wait()
        pltpu.make_async_copy(src(t), inbuf, s_in).wait()
        _transpose_window(inbuf, outbuf)
        pltpu.async_copy(outbuf, dst(t), s_out)
        if issue_next:
            pltpu.async_copy(src(t + 2), inbuf, s_in)

    half(0, in_a, out_a, s_ia, s_oa, True, True)
    half(1, in_b, out_b, s_ib, s_ob, True, True)

    def body(t2, _):
        half(2 * t2, in_a, out_a, s_ia, s_oa, False, True)
        half(2 * t2 + 1, in_b, out_b, s_ib, s_ob, False, True)
        return _

    lax.fori_loop(1, n_t // 2 - 1, body, None)
    half(n_t - 2, in_a, out_a, s_ia, s_oa, False, False)
    half(n_t - 1, in_b, out_b, s_ib, s_ob, False, False)
    pltpu.make_async_copy(out_a, dst(n_t - 2), s_oa).wait()
    pltpu.make_async_copy(out_b, dst(n_t - 1), s_ob).wait()


def _emit_unit(u2, h, idxblk, rows, outbuf):
    iota = _iota16()

    @plsc.parallel_loop(0, UNIT // 16, 1, unroll=4)
    def jgrp(j):
        idxv = idxblk[u2, pl.ds(h * UNIT + 16 * j, 16)]
        qv = (idxv & 3) << 5
        vb = iota + 16 * j
        for c in range(32):
            v = plsc.load_gather(rows, [vb, qv + c])
            outbuf[0, c, pl.ds(16 * j, 16)] = v


def _fill_keys(u2, h, idxblk, keybuf):
    @plsc.parallel_loop(0, UNIT // 16, 1, unroll=8)
    def jgrp(j):
        idxv = idxblk[u2, pl.ds(h * UNIT + 16 * j, 16)]
        keybuf[pl.ds(16 * j, 16)] = idxv >> 2


def _gather_body(scratch, idx_t, out, idxblk, key_a, key_b, rows_a, rows_b,
                 ob_a, ob_b, s_ga, s_gb, s_oa, s_ob):
    wid = lax.axis_index("s") * NC + lax.axis_index("c")
    b0 = wid * B_PER_W
    pltpu.sync_copy(idx_t.at[:, pl.ds(b0, B_PER_W)], idxblk)

    def gather(keybuf, rows, sem):
        pltpu.async_copy(scratch.at[keybuf], rows, sem)

    def out_dst(u2, h):
        return out.at[pl.ds(u2, 1), :, pl.ds(b0 + h * UNIT, UNIT)]

    _fill_keys(0, 0, idxblk, key_a)
    gather(key_a, rows_a, s_ga)
    _fill_keys(0, 1, idxblk, key_b)
    gather(key_b, rows_b, s_gb)

    def half(u2, ph, keybuf, rows, ob, s_g, s_o, first, issue_next):
        if not first:
            pltpu.make_async_copy(ob, out_dst(u2 - 1, ph), s_o).wait()
        pltpu.make_async_copy(scratch.at[keybuf], rows, s_g).wait()
        _emit_unit(u2, ph, idxblk, rows, ob)
        pltpu.async_copy(ob, out_dst(u2, ph), s_o)
        if issue_next:
            _fill_keys(u2 + 1, ph, idxblk, keybuf)
            gather(keybuf, rows, s_g)

    half(0, 0, key_a, rows_a, ob_a, s_ga, s_oa, True, True)
    half(0, 1, key_b, rows_b, ob_b, s_gb, s_ob, True, True)

    def body(u2, _):
        half(u2, 0, key_a, rows_a, ob_a, s_ga, s_oa, False, True)
        half(u2, 1, key_b, rows_b, ob_b, s_gb, s_ob, False, True)
        return _

    lax.fori_loop(1, N_FIELDS - 1, body, None)
    half(N_FIELDS - 1, 0, key_a, rows_a, ob_a, s_ga, s_oa, False, False)
    half(N_FIELDS - 1, 1, key_b, rows_b, ob_b, s_gb, s_ob, False, False)
    pltpu.make_async_copy(ob_a, out_dst(N_FIELDS - 1, 0), s_oa).wait()
    pltpu.make_async_copy(ob_b, out_dst(N_FIELDS - 1, 1), s_ob).wait()


@jax.jit
def _lookup(inputs, embedding):
    emb_t = jnp.transpose(embedding)
    tailc = jnp.reshape(embedding[NWIN * WIN:, :], (16, 128))
    idx_t = jnp.transpose(inputs).astype(jnp.int32)

    mesh = plsc.VectorSubcoreMesh(core_axis_name="c", subcore_axis_name="s")
    params = pltpu.CompilerParams(use_tc_tiling_on_sc=True,
                                  needs_layout_passes=False)

    pack = pl.kernel(
        _pack_body,
        mesh=mesh,
        out_type=jax.ShapeDtypeStruct((SCR_ROWS, 128), jnp.float32),
        scratch_types=[
            pltpu.VMEM((32, WIN), jnp.float32),
            pltpu.VMEM((32, WIN), jnp.float32),
            pltpu.VMEM((WLINES, 128), jnp.float32),
            pltpu.VMEM((WLINES, 128), jnp.float32),
            pltpu.VMEM((16, 128), jnp.float32),
            pltpu.SemaphoreType.DMA,
            pltpu.SemaphoreType.DMA,
            pltpu.SemaphoreType.DMA,
            pltpu.SemaphoreType.DMA,
        ],
        compiler_params=params,
    )
    scratch = pack(emb_t, tailc)

    gather = pl.kernel(
        _gather_body,
        mesh=mesh,
        out_type=jax.ShapeDtypeStruct((N_FIELDS, DIM, BATCH), jnp.float32),
        scratch_types=[
            pltpu.VMEM((N_FIELDS, B_PER_W), jnp.int32),
            pltpu.VMEM((UNIT,), jnp.int32),
            pltpu.VMEM((UNIT,), jnp.int32),
            pltpu.VMEM((UNIT, 128), jnp.float32),
            pltpu.VMEM((UNIT, 128), jnp.float32),
            pltpu.VMEM((1, DIM, UNIT), jnp.float32),
            pltpu.VMEM((1, DIM, UNIT), jnp.float32),
            pltpu.SemaphoreType.DMA,
            pltpu.SemaphoreType.DMA,
            pltpu.SemaphoreType.DMA,
            pltpu.SemaphoreType.DMA,
        ],
        compiler_params=params,
    )
    out = gather(scratch, idx_t)
    return jnp.transpose(out, (2, 0, 1))


def kernel(inputs, embedding):
    return _lookup(inputs, embedding)


# speedup vs baseline: 2.3873x; 2.3873x over previous
import jax
import jax.numpy as jnp
from jax import lax
from jax.experimental import pallas as pl
from jax.experimental.pallas import tpu as pltpu
from jax.experimental.pallas import tpu_sc as plsc

BATCH = 16384
N_FIELDS = 26
DIM = 32
V = 1000000
WIN = 512
NWIN = V // WIN
V_TAIL = V - NWIN * WIN
SCR_ROWS = V // 4
WLINES = WIN // 4

NC = 2
NS = 16
NWK = NC * NS

B_PER_W = BATCH // NWK
UNIT = 256
N_UNITS = N_FIELDS * 2

def _iota16():
    return lax.iota(jnp.int32, 16)


def _transpose_window(inbuf, outbuf):
    iota = _iota16()
    diag = []
    for s in range(16):
        q = (iota + s) & 15
        kq = q >> 2
        for half in range(2):
            cv = iota + 16 * half
            pv = ((q & 3) << 5) + cv
            diag.append((q, kq, cv, pv))

    @plsc.parallel_loop(0, 32, 1, unroll=1)
    def rblk(rb):
        for q, kq, cv, pv in diag:
            v = plsc.load_gather(inbuf, [cv, q + 16 * rb])
            plsc.store_scatter(outbuf, [kq + 4 * rb, pv], v)


def _pack_body(emb_t, tailc, scratch, in_a, in_b, out_a, out_b, tailv,
               s_ia, s_ib, s_oa, s_ob):
    wid = lax.axis_index("s") * NC + lax.axis_index("c")
    last = NWIN - 1

    def win(t):
        return jnp.minimum(wid + NWK * t, last)

    def src(t):
        g = win(t)
        return emb_t.at[:, pl.ds(g * WIN, WIN)]

    def dst(t):
        g = win(t)
        return scratch.at[pl.ds(g * WLINES, WLINES), :]

    pltpu.sync_copy(tailc, tailv)
    pltpu.sync_copy(tailv, scratch.at[pl.ds(SCR_ROWS - 16, 16), :])

    n_t = 62
    pltpu.async_copy(src(0), in_a, s_ia)
    pltpu.async_copy(src(1), in_b, s_ib)

    def half(t, inbuf, outbuf, s_in, s_out, first, issue_next):
        if not first:
            pltpu.make_async_copy(outbuf, dst(t - 2), s_out).wait()
        pltpu.make_async_copy(src(t), inbuf, s_in).wait()
        _transpose_window(inbuf, outbuf)
        pltpu.async_copy(outbuf, dst(t), s_out)
        if issue_next:
            pltpu.async_copy(src(t + 2), inbuf, s_in)

    half(0, in_a, out_a, s_ia, s_oa, True, True)
    half(1, in_b, out_b, s_ib, s_ob, True, True)

    def body(t2, _):
        half(2 * t2, in_a, out_a, s_ia, s_oa, False, True)
        half(2 * t2 + 1, in_b, out_b, s_ib, s_ob, False, True)
        return _

    lax.fori_loop(1, n_t // 2 - 1, body, None)
    half(n_t - 2, in_a, out_a, s_ia, s_oa, False, False)
    half(n_t - 1, in_b, out_b, s_ib, s_ob, False, False)
    pltpu.make_async_copy(out_a, dst(n_t - 2), s_oa).wait()
    pltpu.make_async_copy(out_b, dst(n_t - 1), s_ob).wait()


def _emit_unit(qbuf, rows, outbuf):
    iota = _iota16()
    zero = iota * 0
    diag = []
    for s in range(16):
        q16 = (iota + s) & 15
        for half in range(2):
            cv = iota + 16 * half
            diag.append((q16, cv))

    @plsc.parallel_loop(0, UNIT // 16, 1, unroll=1)
    def jgrp(j):
        for q16, cv in diag:
            bv = q16 + 16 * j
            qg = plsc.load_gather(qbuf, [bv])
            v = plsc.load_gather(rows, [bv, qg + cv])
            plsc.store_scatter(outbuf, [zero, cv, bv], v)


def _fill_keys(u2, h, idxblk, keybuf, qbuf):
    @plsc.parallel_loop(0, UNIT // 16, 1, unroll=8)
    def jgrp(j):
        idxv = idxblk[u2, pl.ds(h * UNIT + 16 * j, 16)]
        keybuf[pl.ds(16 * j, 16)] = idxv >> 2
        qbuf[pl.ds(16 * j, 16)] = (idxv & 3) << 5


def _gather_body(scratch, idx_t, out, idxblk, key_a, key_b, q_a, q_b,
                 rows_a, rows_b, ob_a, ob_b, s_ga, s_gb, s_oa, s_ob):
    wid = lax.axis_index("s") * NC + lax.axis_index("c")
    b0 = wid * B_PER_W
    pltpu.sync_copy(idx_t.at[:, pl.ds(b0, B_PER_W)], idxblk)

    def gather(keybuf, rows, sem):
        pltpu.async_copy(scratch.at[keybuf], rows, sem)

    def out_dst(u2, h):
        return out.at[pl.ds(u2, 1), :, pl.ds(b0 + h * UNIT, UNIT)]

    _fill_keys(0, 0, idxblk, key_a, q_a)
    gather(key_a, rows_a, s_ga)
    _fill_keys(0, 1, idxblk, key_b, q_b)
    gather(key_b, rows_b, s_gb)

    def half(u2, ph, keybuf, qbuf, rows, ob, s_g, s_o, first, issue_next):
        if not first:
            pltpu.make_async_copy(ob, out_dst(u2 - 1, ph), s_o).wait()
        pltpu.make_async_copy(scratch.at[keybuf], rows, s_g).wait()
        _emit_unit(qbuf, rows, ob)
        pltpu.async_copy(ob, out_dst(u2, ph), s_o)
        if issue_next:
            _fill_keys(u2 + 1, ph, idxblk, keybuf, qbuf)
            gather(keybuf, rows, s_g)

    half(0, 0, key_a, q_a, rows_a, ob_a, s_ga, s_oa, True, True)
    half(0, 1, key_b, q_b, rows_b, ob_b, s_gb, s_ob, True, True)

    def body(u2, _):
        half(u2, 0, key_a, q_a, rows_a, ob_a, s_ga, s_oa, False, True)
        half(u2, 1, key_b, q_b, rows_b, ob_b, s_gb, s_ob, False, True)
        return _

    lax.fori_loop(1, N_FIELDS - 1, body, None)
    half(N_FIELDS - 1, 0, key_a, q_a, rows_a, ob_a, s_ga, s_oa, False, False)
    half(N_FIELDS - 1, 1, key_b, q_b, rows_b, ob_b, s_gb, s_ob, False, False)
    pltpu.make_async_copy(ob_a, out_dst(N_FIELDS - 1, 0), s_oa).wait()
    pltpu.make_async_copy(ob_b, out_dst(N_FIELDS - 1, 1), s_ob).wait()


@jax.jit
def _lookup(inputs, embedding):
    emb_t = jnp.transpose(embedding)
    tailc = jnp.reshape(embedding[NWIN * WIN:, :], (16, 128))
    idx_t = jnp.transpose(inputs).astype(jnp.int32)

    mesh = plsc.VectorSubcoreMesh(core_axis_name="c", subcore_axis_name="s")
    params = pltpu.CompilerParams(use_tc_tiling_on_sc=True,
                                  needs_layout_passes=False)

    pack = pl.kernel(
        _pack_body,
        mesh=mesh,
        out_type=jax.ShapeDtypeStruct((SCR_ROWS, 128), jnp.float32),
        scratch_types=[
            pltpu.VMEM((32, WIN), jnp.float32),
            pltpu.VMEM((32, WIN), jnp.float32),
            pltpu.VMEM((WLINES, 128), jnp.float32),
            pltpu.VMEM((WLINES, 128), jnp.float32),
            pltpu.VMEM((16, 128), jnp.float32),
            pltpu.SemaphoreType.DMA,
            pltpu.SemaphoreType.DMA,
            pltpu.SemaphoreType.DMA,
            pltpu.SemaphoreType.DMA,
        ],
        compiler_params=params,
    )
    scratch = pack(emb_t, tailc)

    gather = pl.kernel(
        _gather_body,
        mesh=mesh,
        out_type=jax.ShapeDtypeStruct((N_FIELDS, DIM, BATCH), jnp.float32),
        scratch_types=[
            pltpu.VMEM((N_FIELDS, B_PER_W), jnp.int32),
            pltpu.VMEM((UNIT,), jnp.int32),
            pltpu.VMEM((UNIT,), jnp.int32),
            pltpu.VMEM((UNIT,), jnp.int32),
            pltpu.VMEM((UNIT,), jnp.int32),
            pltpu.VMEM((UNIT, 128), jnp.float32),
            pltpu.VMEM((UNIT, 128), jnp.float32),
            pltpu.VMEM((1, DIM, UNIT), jnp.float32),
            pltpu.VMEM((1, DIM, UNIT), jnp.float32),
            pltpu.SemaphoreType.DMA,
            pltpu.SemaphoreType.DMA,
            pltpu.SemaphoreType.DMA,
            pltpu.SemaphoreType.DMA,
        ],
        compiler_params=params,
    )
    out = gather(scratch, idx_t)
    return jnp.transpose(out, (2, 0, 1))


def kernel(inputs, embedding):
    return _lookup(inputs, embedding)
